# baseline (speedup 1.0000x reference)
import functools

import jax
import jax.numpy as jnp
from jax import lax
from jax.experimental import pallas as pl
from jax.experimental.pallas import tpu as pltpu
from jax.experimental.pallas import tpu_sc as plsc

EPS = 1e-05
MU = 1.0
P_MIN, P_MAX = 0.0, 2.0

_CHUNK = 128
_NSUB = 16
_NQ = 4


def _round_up(x, m):
    return -(-x // m) * m


def _sc_segsum(table, qidx, dst, s_pad, n_pass, chunk=_CHUNK):
    n_chunks = qidx.shape[1]
    sl, ln = table.shape[1], table.shape[2]
    split = n_pass == 0
    if split:
        n_pass = 1
    ngroups = 2 * n_pass if not split else 2
    rpt = s_pad // _NSUB
    nfull, tail = divmod(rpt, _CHUNK)
    zeros = jnp.zeros((_CHUNK, sl, ln), jnp.float32)
    mesh = plsc.VectorSubcoreMesh(core_axis_name="c", subcore_axis_name="s")

    @functools.partial(
        pl.kernel,
        mesh=mesh,
        out_type=jax.ShapeDtypeStruct((ngroups, s_pad, sl, ln), jnp.float32),
        scratch_types=[
            pltpu.VMEM((2, chunk), jnp.int32),
            pltpu.VMEM((2, chunk), jnp.int32),
            pltpu.VMEM((chunk, sl, ln), jnp.float32),
            pltpu.VMEM((chunk, sl, ln), jnp.float32),
            pltpu.VMEM_SHARED((s_pad, sl, ln), jnp.float32),
            pltpu.SemaphoreType.DMA,
            pltpu.SemaphoreType.DMA,
            pltpu.SemaphoreType.DMA,
            pltpu.SemaphoreType.DMA,
        ],
    )
    def k(table_h, qidx_h, dst_h, zero_h, out_h,
          qidxb, dstb, rows0, rows1, acc, semg0, semg1, semi0, semi1):
        c = lax.axis_index("c")
        s = lax.axis_index("s")
        base = s * rpt
        rows = (rows0, rows1)
        semg = (semg0, semg1)
        semi = (semi0, semi1)
        for qi in range(n_pass):
            q = c * n_pass + qi
            if split:
                row = c * _NSUB + s
                drow = row
            else:
                row = q * _NSUB + s
                drow = s
            for z in range(nfull):
                pltpu.sync_copy(zero_h, acc.at[pl.ds(base + z * _CHUNK, _CHUNK)])
            if tail:
                pltpu.sync_copy(zero_h.at[pl.ds(0, tail)],
                                acc.at[pl.ds(base + nfull * _CHUNK, tail)])
            plsc.subcore_barrier()

            pltpu.sync_copy(qidx_h.at[row, 0], qidxb.at[0])
            pltpu.sync_copy(dst_h.at[drow, 0], dstb.at[0])
            pltpu.async_copy(table_h.at[qidxb.at[0]], rows0, semg0)
            pltpu.async_copy(qidx_h.at[row, 1], qidxb.at[1], semi1)
            pltpu.async_copy(dst_h.at[drow, 1], dstb.at[1], semi1)

            def body(g, carry):
                for bb in range(2):
                    j = 2 * g + bb
                    nb = 1 - bb

                    @pl.when(j + 1 < n_chunks)
                    def _():
                        pltpu.make_async_copy(qidx_h.at[row, j + 1],
                                              qidxb.at[nb], semi[nb]).wait()
                        pltpu.make_async_copy(dst_h.at[drow, j + 1],
                                              dstb.at[nb], semi[nb]).wait()
                        pltpu.async_copy(table_h.at[qidxb.at[nb]],
                                         rows[nb], semg[nb])

                    pltpu.make_async_copy(table_h.at[qidxb.at[bb]],
                                          rows[bb], semg[bb]).wait()
                    pltpu.sync_copy(rows[bb], acc.at[dstb.at[bb]], add=True)

                    @pl.when(j + 2 < n_chunks)
                    def _():
                        pltpu.async_copy(qidx_h.at[row, j + 2],
                                         qidxb.at[bb], semi[bb])
                        pltpu.async_copy(dst_h.at[drow, j + 2],
                                         dstb.at[bb], semi[bb])
                return carry

            lax.fori_loop(0, n_chunks // 2, body, 0)
            plsc.subcore_barrier()
            for z in range(nfull):
                pltpu.sync_copy(acc.at[pl.ds(base + z * _CHUNK, _CHUNK)],
                                out_h.at[q, pl.ds(base + z * _CHUNK, _CHUNK)])
            if tail:
                pltpu.sync_copy(acc.at[pl.ds(base + nfull * _CHUNK, tail)],
                                out_h.at[q, pl.ds(base + nfull * _CHUNK, tail)])
            plsc.subcore_barrier()

    return k(table, qidx, dst, zeros)


def _tc_project(X, W, b):
    n, cin = X.shape
    cout = W.shape[1]
    blk = 1000
    grid = n // blk

    def body(x_ref, w_ref, b_ref, xn_ref, mu_ref, mloc):
        i = pl.program_id(0)
        y = jnp.dot(x_ref[...], w_ref[...],
                    preferred_element_type=jnp.float32) + b_ref[...]
        mean = jnp.mean(y, axis=1, keepdims=True)
        var = jnp.mean((y - mean) ** 2, axis=1, keepdims=True)
        xn = (y - mean) / jnp.sqrt(var + EPS)
        xn_ref[...] = xn
        m = jnp.min(xn)

        @pl.when(i == 0)
        def _():
            mloc[0] = m

        @pl.when(i != 0)
        def _():
            mloc[0] = jnp.minimum(mloc[0], m)

        @pl.when(i == grid - 1)
        def _():
            mu_ref[0, 0] = mloc[0] - MU - EPS

    return pl.pallas_call(
        body,
        grid=(grid,),
        in_specs=[pl.BlockSpec((blk, cin), lambda i: (i, 0)),
                  pl.BlockSpec((cin, cout), lambda i: (0, 0)),
                  pl.BlockSpec((1, cout), lambda i: (0, 0))],
        out_specs=[pl.BlockSpec((blk, cout), lambda i: (i, 0)),
                   pl.BlockSpec(memory_space=pltpu.SMEM)],
        out_shape=[jax.ShapeDtypeStruct((n, cout), jnp.float32),
                   jax.ShapeDtypeStruct((1, 1), jnp.float32)],
        scratch_shapes=[pltpu.SMEM((1,), jnp.float32)],
    )(X, W, b.reshape(1, -1))


def _tc_pow_tables(xn, mu, p_param):
    n, c = xn.shape
    h = c // 2
    blk = 1000
    grid = n // blk

    def body(x_ref, m_ref, p_ref, out_ref):
        p = jnp.clip(p_ref[0, 0], P_MIN, P_MAX)
        u = x_ref[...] - m_ref[0, 0]
        t = jnp.exp(p * jnp.log(u))
        ut = u * t
        out_ref[:, 0, :] = ut[:, :h]
        out_ref[:, 1, :] = ut[:, h:]
        out_ref[:, 2, :] = t[:, :h]
        out_ref[:, 3, :] = t[:, h:]

    return pl.pallas_call(
        body,
        grid=(grid,),
        in_specs=[pl.BlockSpec((blk, c), lambda i: (i, 0)),
                  pl.BlockSpec(memory_space=pltpu.SMEM),
                  pl.BlockSpec(memory_space=pltpu.SMEM)],
        out_specs=pl.BlockSpec((blk, _NQ, h), lambda i: (i, 0, 0)),
        out_shape=jax.ShapeDtypeStruct((n, _NQ, h), jnp.float32),
    )(xn, mu, p_param)


def _tc_v2e_epilogue(seg, mu, p_param, e_weight, n_edges):
    h = seg.shape[3]

    def body(seg_ref, m_ref, p_ref, ew_ref, outb_ref, mu2_ref):
        tot = seg_ref[0] + seg_ref[1]
        num = jnp.concatenate([tot[:, 0, :], tot[:, 1, :]], axis=1)
        den = jnp.concatenate([tot[:, 2, :], tot[:, 3, :]], axis=1)
        xe = (num / (den + EPS) + m_ref[0, 0]) * ew_ref[...]
        mu2 = jnp.min(xe) - MU - EPS
        p = jnp.clip(p_ref[0, 0], P_MIN, P_MAX)
        u = xe - mu2
        t = jnp.exp(p * jnp.log(u))
        ut = u * t
        outb_ref[0] = ut[:, :h]
        outb_ref[1] = ut[:, h:]
        outb_ref[2] = t[:, :h]
        outb_ref[3] = t[:, h:]
        mu2_ref[0, 0] = mu2

    return pl.pallas_call(
        body,
        grid=(1,),
        in_specs=[pl.BlockSpec((2, n_edges, _NQ, h), lambda i: (0, 0, 0, 0)),
                  pl.BlockSpec(memory_space=pltpu.SMEM),
                  pl.BlockSpec(memory_space=pltpu.SMEM),
                  pl.BlockSpec((n_edges, 1), lambda i: (0, 0))],
        out_specs=[pl.BlockSpec((_NQ, n_edges, h), lambda i: (0, 0, 0)),
                   pl.BlockSpec(memory_space=pltpu.SMEM)],
        out_shape=[jax.ShapeDtypeStruct((_NQ, n_edges, h), jnp.float32),
                   jax.ShapeDtypeStruct((1, 1), jnp.float32)],
    )(seg, mu, p_param, e_weight.reshape(-1, 1))


def _tc_e2v_epilogue(seg2, mu2, n_nodes):
    h = seg2.shape[2]
    c = 2 * h
    blk = 1000
    grid = n_nodes // blk

    def body(seg_ref, m_ref, xv_ref, st_ref, acc):
        i = pl.program_id(0)
        num = jnp.concatenate([seg_ref[0], seg_ref[1]], axis=1)
        den = jnp.concatenate([seg_ref[2], seg_ref[3]], axis=1)
        xv = jnp.maximum(num / (den + EPS) + m_ref[0, 0], 0.0)
        xv_ref[...] = xv

        @pl.when(i == 0)
        def _():
            acc[...] = jnp.zeros_like(acc)

        acc[0:1, :] += jnp.sum(xv, axis=0, keepdims=True)
        acc[1:2, :] += jnp.sum(xv * xv, axis=0, keepdims=True)

        @pl.when(i == grid - 1)
        def _():
            st_ref[...] = acc[...]

    return pl.pallas_call(
        body,
        grid=(grid,),
        in_specs=[pl.BlockSpec((_NQ, blk, h), lambda i: (0, i, 0)),
                  pl.BlockSpec(memory_space=pltpu.SMEM)],
        out_specs=[pl.BlockSpec((blk, c), lambda i: (i, 0)),
                   pl.BlockSpec((8, c), lambda i: (0, 0))],
        out_shape=[jax.ShapeDtypeStruct((n_nodes, c), jnp.float32),
                   jax.ShapeDtypeStruct((8, c), jnp.float32)],
        scratch_shapes=[pltpu.VMEM((8, c), jnp.float32)],
    )(seg2, mu2)


def _tc_batchnorm(xv, stats, gamma, beta):
    n, c = xv.shape
    blk = 1000
    grid = n // blk
    inv_n = 1.0 / n

    def body(xv_ref, st_ref, g_ref, b_ref, out_ref):
        mean = st_ref[0:1, :] * inv_n
        var = jnp.maximum(st_ref[1:2, :] * inv_n - mean * mean, 0.0)
        out_ref[...] = (g_ref[...] * (xv_ref[...] - mean)
                        / jnp.sqrt(var + 1e-05) + b_ref[...])

    return pl.pallas_call(
        body,
        grid=(grid,),
        in_specs=[pl.BlockSpec((blk, c), lambda i: (i, 0)),
                  pl.BlockSpec((8, c), lambda i: (0, 0)),
                  pl.BlockSpec((1, c), lambda i: (0, 0)),
                  pl.BlockSpec((1, c), lambda i: (0, 0))],
        out_specs=pl.BlockSpec((blk, c), lambda i: (i, 0)),
        out_shape=jax.ShapeDtypeStruct((n, c), jnp.float32),
    )(xv, stats, gamma.reshape(1, -1), beta.reshape(1, -1))


def kernel(X, node_idx, edge_idx, W, b, p_param, gamma, beta, e_weight):
    n_nodes, _ = X.shape
    n_edges = e_weight.shape[0]
    nnz = node_idx.shape[0]

    group = _NSUB * _CHUNK * 4
    nnz_pad = _round_up(nnz, group)
    pad = nnz_pad - nnz
    n_chunks = nnz_pad // (_NSUB * _CHUNK)
    zpad = jnp.zeros((pad,), jnp.int32)
    ni_src = jnp.concatenate([node_idx, zpad])
    ei_src = jnp.concatenate([edge_idx, zpad])
    ni_dst = jnp.concatenate([node_idx, jnp.full((pad,), n_nodes, jnp.int32)])
    ei_dst = jnp.concatenate([edge_idx, jnp.full((pad,), n_edges, jnp.int32)])

    chunk_v = 64
    n_chunks_v = nnz_pad // (2 * _NSUB * chunk_v)
    q4 = jnp.arange(_NQ, dtype=jnp.int32)[:, None]
    qidx_v2e = ni_src.reshape(2 * _NSUB, n_chunks_v, chunk_v)
    dst_v2e = ei_dst.reshape(2 * _NSUB, n_chunks_v, chunk_v)
    qidx_e2v = (q4 * n_edges + ei_src[None, :]).reshape(
        _NQ * _NSUB, n_chunks, _CHUNK)
    dst_e2v = ni_dst.reshape(_NSUB, n_chunks, _CHUNK)

    s_pad_e = _round_up(n_edges + 1, _NSUB)
    s_pad_n = _round_up(n_nodes + 1, _NSUB)

    xn, mu = _tc_project(X, W, b)
    a_tab = _tc_pow_tables(xn, mu, p_param)
    h = a_tab.shape[2]
    seg1 = _sc_segsum(a_tab, qidx_v2e, dst_v2e, s_pad_e, n_pass=0,
                      chunk=chunk_v)
    b_tab, mu2 = _tc_v2e_epilogue(seg1, mu, p_param, e_weight, n_edges)
    seg2 = _sc_segsum(b_tab.reshape(_NQ * n_edges, 1, h), qidx_e2v, dst_e2v,
                      s_pad_n, n_pass=2)
    xv, stats = _tc_e2v_epilogue(seg2.reshape(_NQ, s_pad_n, h), mu2, n_nodes)
    return _tc_batchnorm(xv, stats, gamma, beta)


# speedup vs baseline: 1.0096x; 1.0096x over previous
import functools

import jax
import jax.numpy as jnp
from jax import lax
from jax.experimental import pallas as pl
from jax.experimental.pallas import tpu as pltpu
from jax.experimental.pallas import tpu_sc as plsc

EPS = 1e-05
MU = 1.0
P_MIN, P_MAX = 0.0, 2.0

_CHUNK = 128
_NSUB = 16
_NQ = 4


def _round_up(x, m):
    return -(-x // m) * m


def _sc_segsum(table, qidx, dst, s_pad, n_pass, chunk=_CHUNK):
    n_chunks = qidx.shape[1]
    sl, ln = table.shape[1], table.shape[2]
    split = n_pass == 0
    if split:
        n_pass = 1
    ngroups = 2 * n_pass if not split else 2
    rpt = s_pad // _NSUB
    nfull, tail = divmod(rpt, _CHUNK)
    zeros = jnp.zeros((_CHUNK, sl, ln), jnp.float32)
    mesh = plsc.VectorSubcoreMesh(core_axis_name="c", subcore_axis_name="s")

    @functools.partial(
        pl.kernel,
        mesh=mesh,
        out_type=jax.ShapeDtypeStruct((ngroups, s_pad, sl, ln), jnp.float32),
        scratch_types=[
            pltpu.VMEM((2, chunk), jnp.int32),
            pltpu.VMEM((2, chunk), jnp.int32),
            pltpu.VMEM((chunk, sl, ln), jnp.float32),
            pltpu.VMEM((chunk, sl, ln), jnp.float32),
            pltpu.VMEM_SHARED((s_pad, sl, ln), jnp.float32),
            pltpu.SemaphoreType.DMA,
            pltpu.SemaphoreType.DMA,
            pltpu.SemaphoreType.DMA,
            pltpu.SemaphoreType.DMA,
        ],
    )
    def k(table_h, qidx_h, dst_h, zero_h, out_h,
          qidxb, dstb, rows0, rows1, acc, semg0, semg1, semi0, semi1):
        c = lax.axis_index("c")
        s = lax.axis_index("s")
        base = s * rpt
        rows = (rows0, rows1)
        semg = (semg0, semg1)
        semi = (semi0, semi1)
        for qi in range(n_pass):
            q = c * n_pass + qi
            if split:
                row = c * _NSUB + s
                drow = row
            else:
                row = q * _NSUB + s
                drow = s
            for z in range(nfull):
                pltpu.sync_copy(zero_h, acc.at[pl.ds(base + z * _CHUNK, _CHUNK)])
            if tail:
                pltpu.sync_copy(zero_h.at[pl.ds(0, tail)],
                                acc.at[pl.ds(base + nfull * _CHUNK, tail)])
            plsc.subcore_barrier()

            pltpu.sync_copy(qidx_h.at[row, 0], qidxb.at[0])
            pltpu.sync_copy(dst_h.at[drow, 0], dstb.at[0])
            pltpu.async_copy(table_h.at[qidxb.at[0]], rows0, semg0)
            pltpu.async_copy(qidx_h.at[row, 1], qidxb.at[1], semi1)
            pltpu.async_copy(dst_h.at[drow, 1], dstb.at[1], semi1)

            def body(g, carry):
                for bb in range(2):
                    j = 2 * g + bb
                    nb = 1 - bb

                    @pl.when(j + 1 < n_chunks)
                    def _():
                        pltpu.make_async_copy(qidx_h.at[row, j + 1],
                                              qidxb.at[nb], semi[nb]).wait()
                        pltpu.make_async_copy(dst_h.at[drow, j + 1],
                                              dstb.at[nb], semi[nb]).wait()
                        pltpu.async_copy(table_h.at[qidxb.at[nb]],
                                         rows[nb], semg[nb])

                    pltpu.make_async_copy(table_h.at[qidxb.at[bb]],
                                          rows[bb], semg[bb]).wait()
                    pltpu.sync_copy(rows[bb], acc.at[dstb.at[bb]], add=True)

                    @pl.when(j + 2 < n_chunks)
                    def _():
                        pltpu.async_copy(qidx_h.at[row, j + 2],
                                         qidxb.at[bb], semi[bb])
                        pltpu.async_copy(dst_h.at[drow, j + 2],
                                         dstb.at[bb], semi[bb])
                return carry

            lax.fori_loop(0, n_chunks // 2, body, 0)
            plsc.subcore_barrier()
            for z in range(nfull):
                pltpu.sync_copy(acc.at[pl.ds(base + z * _CHUNK, _CHUNK)],
                                out_h.at[q, pl.ds(base + z * _CHUNK, _CHUNK)])
            if tail:
                pltpu.sync_copy(acc.at[pl.ds(base + nfull * _CHUNK, tail)],
                                out_h.at[q, pl.ds(base + nfull * _CHUNK, tail)])
            plsc.subcore_barrier()

    return k(table, qidx, dst, zeros)


def _tc_project(X, W, b):
    n, cin = X.shape
    cout = W.shape[1]
    blk = 1000
    grid = n // blk

    def body(x_ref, w_ref, b_ref, xn_ref, mu_ref, mloc):
        i = pl.program_id(0)
        y = jnp.dot(x_ref[...], w_ref[...],
                    preferred_element_type=jnp.float32) + b_ref[...]
        mean = jnp.mean(y, axis=1, keepdims=True)
        var = jnp.mean((y - mean) ** 2, axis=1, keepdims=True)
        xn = (y - mean) / jnp.sqrt(var + EPS)
        xn_ref[...] = xn
        m = jnp.min(xn)

        @pl.when(i == 0)
        def _():
            mloc[0] = m

        @pl.when(i != 0)
        def _():
            mloc[0] = jnp.minimum(mloc[0], m)

        @pl.when(i == grid - 1)
        def _():
            mu_ref[0, 0] = mloc[0] - MU - EPS

    return pl.pallas_call(
        body,
        grid=(grid,),
        in_specs=[pl.BlockSpec((blk, cin), lambda i: (i, 0)),
                  pl.BlockSpec((cin, cout), lambda i: (0, 0)),
                  pl.BlockSpec((1, cout), lambda i: (0, 0))],
        out_specs=[pl.BlockSpec((blk, cout), lambda i: (i, 0)),
                   pl.BlockSpec(memory_space=pltpu.SMEM)],
        out_shape=[jax.ShapeDtypeStruct((n, cout), jnp.float32),
                   jax.ShapeDtypeStruct((1, 1), jnp.float32)],
        scratch_shapes=[pltpu.SMEM((1,), jnp.float32)],
    )(X, W, b.reshape(1, -1))


def _tc_pow_tables(xn, mu, p_param):
    n, c = xn.shape
    h = c // 2
    blk = 1000
    grid = n // blk

    def body(x_ref, m_ref, p_ref, out_ref):
        p = jnp.clip(p_ref[0, 0], P_MIN, P_MAX)
        u = x_ref[...] - m_ref[0, 0]
        t = jnp.exp(p * jnp.log(u))
        ut = u * t
        out_ref[0] = ut[:, :h]
        out_ref[1] = ut[:, h:]
        out_ref[2] = t[:, :h]
        out_ref[3] = t[:, h:]

    return pl.pallas_call(
        body,
        grid=(grid,),
        in_specs=[pl.BlockSpec((blk, c), lambda i: (i, 0)),
                  pl.BlockSpec(memory_space=pltpu.SMEM),
                  pl.BlockSpec(memory_space=pltpu.SMEM)],
        out_specs=pl.BlockSpec((_NQ, blk, h), lambda i: (0, i, 0)),
        out_shape=jax.ShapeDtypeStruct((_NQ, n, h), jnp.float32),
    )(xn, mu, p_param)


def _tc_v2e_epilogue(seg, mu, p_param, e_weight, n_edges):
    h = seg.shape[3]

    def body(seg_ref, m_ref, p_ref, ew_ref, outb_ref, mu2_ref):
        num = jnp.concatenate([seg_ref[0, :, 0, :], seg_ref[1, :, 0, :]],
                              axis=1)
        den = jnp.concatenate([seg_ref[0, :, 1, :], seg_ref[1, :, 1, :]],
                              axis=1)
        xe = (num / (den + EPS) + m_ref[0, 0]) * ew_ref[...]
        mu2 = jnp.min(xe) - MU - EPS
        p = jnp.clip(p_ref[0, 0], P_MIN, P_MAX)
        u = xe - mu2
        t = jnp.exp(p * jnp.log(u))
        ut = u * t
        outb_ref[0] = ut[:, :h]
        outb_ref[1] = ut[:, h:]
        outb_ref[2] = t[:, :h]
        outb_ref[3] = t[:, h:]
        mu2_ref[0, 0] = mu2

    return pl.pallas_call(
        body,
        grid=(1,),
        in_specs=[pl.BlockSpec((2, n_edges, 2, h), lambda i: (0, 0, 0, 0)),
                  pl.BlockSpec(memory_space=pltpu.SMEM),
                  pl.BlockSpec(memory_space=pltpu.SMEM),
                  pl.BlockSpec((n_edges, 1), lambda i: (0, 0))],
        out_specs=[pl.BlockSpec((_NQ, n_edges, h), lambda i: (0, 0, 0)),
                   pl.BlockSpec(memory_space=pltpu.SMEM)],
        out_shape=[jax.ShapeDtypeStruct((_NQ, n_edges, h), jnp.float32),
                   jax.ShapeDtypeStruct((1, 1), jnp.float32)],
    )(seg, mu, p_param, e_weight.reshape(-1, 1))


def _tc_e2v_epilogue(seg2, mu2, n_nodes):
    h = seg2.shape[2]
    c = 2 * h
    blk = 1000
    grid = n_nodes // blk

    def body(seg_ref, m_ref, xv_ref, st_ref, acc):
        i = pl.program_id(0)
        num = jnp.concatenate([seg_ref[0], seg_ref[1]], axis=1)
        den = jnp.concatenate([seg_ref[2], seg_ref[3]], axis=1)
        xv = jnp.maximum(num / (den + EPS) + m_ref[0, 0], 0.0)
        xv_ref[...] = xv

        @pl.when(i == 0)
        def _():
            acc[...] = jnp.zeros_like(acc)

        acc[0:1, :] += jnp.sum(xv, axis=0, keepdims=True)
        acc[1:2, :] += jnp.sum(xv * xv, axis=0, keepdims=True)

        @pl.when(i == grid - 1)
        def _():
            st_ref[...] = acc[...]

    return pl.pallas_call(
        body,
        grid=(grid,),
        in_specs=[pl.BlockSpec((_NQ, blk, h), lambda i: (0, i, 0)),
                  pl.BlockSpec(memory_space=pltpu.SMEM)],
        out_specs=[pl.BlockSpec((blk, c), lambda i: (i, 0)),
                   pl.BlockSpec((8, c), lambda i: (0, 0))],
        out_shape=[jax.ShapeDtypeStruct((n_nodes, c), jnp.float32),
                   jax.ShapeDtypeStruct((8, c), jnp.float32)],
        scratch_shapes=[pltpu.VMEM((8, c), jnp.float32)],
    )(seg2, mu2)


def _tc_batchnorm(xv, stats, gamma, beta):
    n, c = xv.shape
    blk = 1000
    grid = n // blk
    inv_n = 1.0 / n

    def body(xv_ref, st_ref, g_ref, b_ref, out_ref):
        mean = st_ref[0:1, :] * inv_n
        var = jnp.maximum(st_ref[1:2, :] * inv_n - mean * mean, 0.0)
        out_ref[...] = (g_ref[...] * (xv_ref[...] - mean)
                        / jnp.sqrt(var + 1e-05) + b_ref[...])

    return pl.pallas_call(
        body,
        grid=(grid,),
        in_specs=[pl.BlockSpec((blk, c), lambda i: (i, 0)),
                  pl.BlockSpec((8, c), lambda i: (0, 0)),
                  pl.BlockSpec((1, c), lambda i: (0, 0)),
                  pl.BlockSpec((1, c), lambda i: (0, 0))],
        out_specs=pl.BlockSpec((blk, c), lambda i: (i, 0)),
        out_shape=jax.ShapeDtypeStruct((n, c), jnp.float32),
    )(xv, stats, gamma.reshape(1, -1), beta.reshape(1, -1))


def kernel(X, node_idx, edge_idx, W, b, p_param, gamma, beta, e_weight):
    n_nodes, _ = X.shape
    n_edges = e_weight.shape[0]
    nnz = node_idx.shape[0]

    group = _NSUB * _CHUNK * 4
    nnz_pad = _round_up(nnz, group)
    pad = nnz_pad - nnz
    n_chunks = nnz_pad // (_NSUB * _CHUNK)
    zpad = jnp.zeros((pad,), jnp.int32)
    ni_src = jnp.concatenate([node_idx, zpad])
    ei_src = jnp.concatenate([edge_idx, zpad])
    ni_dst = jnp.concatenate([node_idx, jnp.full((pad,), n_nodes, jnp.int32)])
    ei_dst = jnp.concatenate([edge_idx, jnp.full((pad,), n_edges, jnp.int32)])

    q2 = jnp.arange(2, dtype=jnp.int32)[:, None]
    q4 = jnp.arange(_NQ, dtype=jnp.int32)[:, None]
    qidx_v2e = (q2 * n_nodes + ni_src[None, :]).reshape(
        2 * _NSUB, n_chunks, _CHUNK)
    dst_v2e = ei_dst.reshape(_NSUB, n_chunks, _CHUNK)
    qidx_e2v = (q4 * n_edges + ei_src[None, :]).reshape(
        _NQ * _NSUB, n_chunks, _CHUNK)
    dst_e2v = ni_dst.reshape(_NSUB, n_chunks, _CHUNK)

    s_pad_e = _round_up(n_edges + 1, _NSUB)
    s_pad_n = _round_up(n_nodes + 1, _NSUB)

    xn, mu = _tc_project(X, W, b)
    a_tab = _tc_pow_tables(xn, mu, p_param)
    h = a_tab.shape[2]
    a2 = jnp.stack([a_tab[0:2], a_tab[2:4]], axis=2).reshape(
        2 * n_nodes, 2, h)
    seg1 = _sc_segsum(a2, qidx_v2e, dst_v2e, s_pad_e, n_pass=1)
    b_tab, mu2 = _tc_v2e_epilogue(seg1, mu, p_param, e_weight, n_edges)
    seg2 = _sc_segsum(b_tab.reshape(_NQ * n_edges, 1, h), qidx_e2v, dst_e2v,
                      s_pad_n, n_pass=2)
    xv, stats = _tc_e2v_epilogue(seg2.reshape(_NQ, s_pad_n, h), mu2, n_nodes)
    return _tc_batchnorm(xv, stats, gamma, beta)
